# Initial kernel scaffold; baseline (speedup 1.0000x reference)
import jax
import jax.numpy as jnp
from jax.experimental import pallas as pl


def kernel(hidden_states, selected_experts, routing_weights, wi, wo):
    raise NotImplementedError("write your pallas kernel here")


# speedup vs baseline: 1.1548x; 1.1548x over previous
import jax
import jax.numpy as jnp
from jax.experimental import pallas as pl


def _expert_mlp_kernel(x_ref, wi_ref, wo_ref, m0_ref, m1_ref, r0_ref, r1_ref,
                       o_ref, *, bt):
    e = pl.program_id(0)
    f = pl.program_id(1)
    t = pl.program_id(2)

    x = x_ref[...].astype(jnp.bfloat16)
    wib = wi_ref[0].astype(jnp.bfloat16)
    h = jax.lax.dot_general(x, wib, (((1,), (1,)), ((), ())),
                            preferred_element_type=jnp.float32)
    h = jnp.maximum(h, 0.0).astype(jnp.bfloat16)
    wob = wo_ref[0].astype(jnp.bfloat16)
    o = jax.lax.dot_general(h, wob, (((1,), (1,)), ((), ())),
                            preferred_element_type=jnp.float32)

    call = m0_ref[...] * r0_ref[...] + m1_ref[...] * r1_ref[...]
    onehot = (jax.lax.broadcasted_iota(jnp.int32, call.shape, 1) == e)
    c = jnp.sum(jnp.where(onehot, call, 0.0), axis=1, keepdims=True)
    contrib = o * c

    rows = pl.ds(t * bt, bt)
    first = (e == 0) & (f == 0)

    @pl.when(first)
    def _():
        o_ref[rows, :] = contrib

    @pl.when(jnp.logical_not(first))
    def _():
        o_ref[rows, :] += contrib


def kernel(hidden_states, selected_experts, routing_weights, wi, wo):
    T, D = hidden_states.shape
    E, F, _ = wi.shape

    maskf = selected_experts.astype(jnp.float32)
    m0 = maskf[:, 0, :]
    m1 = maskf[:, 1, :]
    r0 = routing_weights[:, 0:1]
    r1 = routing_weights[:, 1:2]

    BT = 512
    BF = 1536
    NT = T // BT
    NF = F // BF

    import functools
    body = functools.partial(_expert_mlp_kernel, bt=BT)

    out = pl.pallas_call(
        body,
        grid=(E, NF, NT),
        in_specs=[
            pl.BlockSpec((BT, D), lambda e, f, t: (t, 0)),
            pl.BlockSpec((1, BF, D), lambda e, f, t: (e, f, 0)),
            pl.BlockSpec((1, D, BF), lambda e, f, t: (e, 0, f)),
            pl.BlockSpec((BT, E), lambda e, f, t: (t, 0)),
            pl.BlockSpec((BT, E), lambda e, f, t: (t, 0)),
            pl.BlockSpec((BT, 1), lambda e, f, t: (t, 0)),
            pl.BlockSpec((BT, 1), lambda e, f, t: (t, 0)),
        ],
        out_specs=pl.BlockSpec((T, D), lambda e, f, t: (0, 0)),
        out_shape=jax.ShapeDtypeStruct((T, D), jnp.float32),
    )(hidden_states, wi, wo, m0, m1, r0, r1)
    return out
